# baseline (speedup 1.0000x reference)
import functools

import jax
import jax.numpy as jnp
from jax import lax
from jax.experimental import pallas as pl
from jax.experimental.pallas import tpu as pltpu
from jax.experimental.pallas import tpu_sc as plsc

NUM_CORES = 2
NUM_SUBCORES = 16
NUM_WORKERS = NUM_CORES * NUM_SUBCORES
B = 16384
EMB = 64
WIDE = 2 * EMB
CHUNK = 128
B_PER_W = B // NUM_WORKERS
N_CHUNKS = B_PER_W // CHUNK
LANES = 16


def _gather3_body(users_hbm, items_hbm, neg_hbm, u_tab, v_tab,
                  out_u, out_v, out_n,
                  idx_raw, idx_hi, rows, idx_sem, gat_sem, out_sem):
    wid = lax.axis_index("s") * NUM_CORES + lax.axis_index("c")
    base = wid * B_PER_W

    tables = ((users_hbm, u_tab, out_u), (items_hbm, v_tab, out_v),
              (neg_hbm, v_tab, out_n))

    for src_idx, tab, dst in tables:
        idx_copies = []
        for j in range(N_CHUNKS):
            c = pltpu.make_async_copy(
                src_idx.at[pl.ds(base + j * CHUNK, CHUNK)],
                idx_raw.at[j], idx_sem)
            c.start()
            idx_copies.append(c)
        for c in idx_copies:
            c.wait()

        for j in range(N_CHUNKS):
            for v in range(CHUNK // LANES):
                sl = pl.ds(v * LANES, LANES)
                idx_hi[j, sl] = lax.shift_right_logical(
                    idx_raw[j, sl], jnp.int32(1))

        gathers = []
        for j in range(N_CHUNKS):
            c = pltpu.make_async_copy(tab.at[idx_hi.at[j]], rows.at[j],
                                      gat_sem)
            c.start()
            gathers.append(c)
        for c in gathers:
            c.wait()

        outs = []
        for j in range(N_CHUNKS):
            c = pltpu.make_async_copy(
                rows.at[j], dst.at[pl.ds(base + j * CHUNK, CHUNK)],
                out_sem)
            c.start()
            outs.append(c)
        for c in outs:
            c.wait()


@jax.jit
def kernel(users, items, neg_items, U, V):
    mesh = plsc.VectorSubcoreMesh(core_axis_name="c", subcore_axis_name="s",
                                  num_cores=NUM_CORES,
                                  num_subcores=NUM_SUBCORES)
    wide_sd = jax.ShapeDtypeStruct((B, WIDE), jnp.float32)
    f = pl.kernel(
        _gather3_body,
        out_type=(wide_sd, wide_sd, wide_sd),
        mesh=mesh,
        compiler_params=pltpu.CompilerParams(use_tc_tiling_on_sc=True),
        scratch_types=[
            pltpu.VMEM((N_CHUNKS, CHUNK), jnp.int32),
            pltpu.VMEM((N_CHUNKS, CHUNK), jnp.int32),
            pltpu.VMEM((N_CHUNKS, CHUNK, WIDE), jnp.float32),
            pltpu.SemaphoreType.DMA,
            pltpu.SemaphoreType.DMA,
            pltpu.SemaphoreType.DMA,
        ],
    )
    def pair(tab):
        return jnp.concatenate([tab[0::2], tab[1::2]], axis=1)

    u_wide, v_wide, n_wide = f(users, items, neg_items, pair(U), pair(V))

    def half(wide, idx):
        odd = (idx & 1)[:, None] == 1
        return jnp.where(odd, wide[:, EMB:], wide[:, :EMB])

    return (half(u_wide, users), half(v_wide, items),
            half(n_wide, neg_items))


# speedup vs baseline: 13.4939x; 13.4939x over previous
import functools

import jax
import jax.numpy as jnp
from jax import lax
from jax.experimental import pallas as pl
from jax.experimental.pallas import tpu as pltpu
from jax.experimental.pallas import tpu_sc as plsc

NUM_CORES = 2
NUM_SUBCORES = 16
NUM_WORKERS = NUM_CORES * NUM_SUBCORES
B = 16384
EMB = 64
WIDE = 2 * EMB
CHUNK = 128
B_PER_W = B // NUM_WORKERS
N_CHUNKS = B_PER_W // CHUNK
LANES = 16


def _gather3_body(users_hbm, items_hbm, neg_hbm, u_tab, v_tab,
                  out_u, out_v, out_n,
                  idx_raw, idx_hi, rows, idx_sem, gat_sem, out_sem):
    wid = lax.axis_index("s") * NUM_CORES + lax.axis_index("c")
    base = wid * B_PER_W

    tables = ((users_hbm, u_tab, out_u), (items_hbm, v_tab, out_v),
              (neg_hbm, v_tab, out_n))

    for src_idx, tab, dst in tables:
        idx_copies = []
        for j in range(N_CHUNKS):
            c = pltpu.make_async_copy(
                src_idx.at[pl.ds(base + j * CHUNK, CHUNK)],
                idx_raw.at[j], idx_sem)
            c.start()
            idx_copies.append(c)
        for c in idx_copies:
            c.wait()

        for j in range(N_CHUNKS):
            for v in range(CHUNK // LANES):
                sl = pl.ds(v * LANES, LANES)
                idx_hi[j, sl] = lax.shift_right_logical(
                    idx_raw[j, sl], jnp.int32(1))

        gathers = []
        for j in range(N_CHUNKS):
            c = pltpu.make_async_copy(tab.at[idx_hi.at[j]], rows.at[j],
                                      gat_sem)
            c.start()
            gathers.append(c)
        for c in gathers:
            c.wait()

        outs = []
        for j in range(N_CHUNKS):
            c = pltpu.make_async_copy(
                rows.at[j], dst.at[pl.ds(base + j * CHUNK, CHUNK)],
                out_sem)
            c.start()
            outs.append(c)
        for c in outs:
            c.wait()


@jax.jit
def kernel(users, items, neg_items, U, V):
    mesh = plsc.VectorSubcoreMesh(core_axis_name="c", subcore_axis_name="s",
                                  num_cores=NUM_CORES,
                                  num_subcores=NUM_SUBCORES)
    wide_sd = jax.ShapeDtypeStruct((B, WIDE), jnp.float32)
    f = pl.kernel(
        _gather3_body,
        out_type=(wide_sd, wide_sd, wide_sd),
        mesh=mesh,
        compiler_params=pltpu.CompilerParams(use_tc_tiling_on_sc=True),
        scratch_types=[
            pltpu.VMEM((N_CHUNKS, CHUNK), jnp.int32),
            pltpu.VMEM((N_CHUNKS, CHUNK), jnp.int32),
            pltpu.VMEM((N_CHUNKS, CHUNK, WIDE), jnp.float32),
            pltpu.SemaphoreType.DMA,
            pltpu.SemaphoreType.DMA,
            pltpu.SemaphoreType.DMA,
        ],
    )
    u_wide, v_wide, n_wide = f(
        users, items, neg_items,
        U.reshape(U.shape[0] // 2, WIDE), V.reshape(V.shape[0] // 2, WIDE))

    def half(wide, idx):
        odd = (idx & 1)[:, None] == 1
        return jnp.where(odd, wide[:, EMB:], wide[:, :EMB])

    return (half(u_wide, users), half(v_wide, items),
            half(n_wide, neg_items))


# speedup vs baseline: 22.4985x; 1.6673x over previous
import jax
import jax.numpy as jnp
from jax import lax
from jax.experimental import pallas as pl
from jax.experimental.pallas import tpu as pltpu
from jax.experimental.pallas import tpu_sc as plsc

NUM_CORES = 2
NUM_SUBCORES = 16
NUM_WORKERS = NUM_CORES * NUM_SUBCORES
B = 16384
EMB = 64
CHUNK = 128
B_PER_W = B // NUM_WORKERS
N_CHUNKS = B_PER_W // CHUNK


def _gather3_body(users_hbm, items_hbm, neg_hbm, u_tab, v_tab,
                  out_u, out_v, out_n,
                  idx_raw, rows, idx_sem, gat_sem, out_sem):
    wid = lax.axis_index("s") * NUM_CORES + lax.axis_index("c")
    base = wid * B_PER_W

    tables = ((users_hbm, u_tab, out_u), (items_hbm, v_tab, out_v),
              (neg_hbm, v_tab, out_n))

    for src_idx, tab, dst in tables:
        idx_copies = []
        for j in range(N_CHUNKS):
            c = pltpu.make_async_copy(
                src_idx.at[pl.ds(base + j * CHUNK, CHUNK)],
                idx_raw.at[j], idx_sem)
            c.start()
            idx_copies.append(c)
        for c in idx_copies:
            c.wait()

        def issue(g, j):
            vec = idx_raw[j, pl.ds(g * 16, 16)]
            for k in range(16):
                pltpu.make_async_copy(
                    tab.at[vec[k]], rows.at[j, g * 16 + k], gat_sem).start()
            return j

        for j in range(N_CHUNKS):
            lax.fori_loop(0, CHUNK // 16, issue, j)

        def drain(r, carry):
            pltpu.make_async_copy(tab.at[0], rows.at[0, 0], gat_sem).wait()
            return carry

        lax.fori_loop(0, N_CHUNKS * CHUNK, drain, 0)

        outs = []
        for j in range(N_CHUNKS):
            c = pltpu.make_async_copy(
                rows.at[j], dst.at[pl.ds(base + j * CHUNK, CHUNK)],
                out_sem)
            c.start()
            outs.append(c)
        for c in outs:
            c.wait()


@jax.jit
def kernel(users, items, neg_items, U, V):
    mesh = plsc.VectorSubcoreMesh(core_axis_name="c", subcore_axis_name="s",
                                  num_cores=NUM_CORES,
                                  num_subcores=NUM_SUBCORES)
    out_sd = jax.ShapeDtypeStruct((B, EMB), jnp.float32)
    f = pl.kernel(
        _gather3_body,
        out_type=(out_sd, out_sd, out_sd),
        mesh=mesh,
        compiler_params=pltpu.CompilerParams(use_tc_tiling_on_sc=True),
        scratch_types=[
            pltpu.VMEM((N_CHUNKS, CHUNK), jnp.int32),
            pltpu.VMEM((N_CHUNKS, CHUNK, EMB), jnp.float32),
            pltpu.SemaphoreType.DMA,
            pltpu.SemaphoreType.DMA,
            pltpu.SemaphoreType.DMA,
        ],
    )
    return f(users, items, neg_items, U, V)


# speedup vs baseline: 22.6935x; 1.0087x over previous
import jax
import jax.numpy as jnp
from jax import lax
from jax.experimental import pallas as pl
from jax.experimental.pallas import tpu as pltpu
from jax.experimental.pallas import tpu_sc as plsc

NUM_CORES = 2
NUM_SUBCORES = 16
NUM_WORKERS = NUM_CORES * NUM_SUBCORES
B = 16384
EMB = 64
CHUNK = 128
B_PER_W = B // NUM_WORKERS
N_CHUNKS = B_PER_W // CHUNK


def _gather3_body(users_hbm, items_hbm, neg_hbm, u_tab, v_tab,
                  out_u, out_v, out_n,
                  idx_raw, rows, idx_sem, gat_sem, out_sem):
    wid = lax.axis_index("s") * NUM_CORES + lax.axis_index("c")
    base = wid * B_PER_W

    tables = ((users_hbm, u_tab, out_u), (items_hbm, v_tab, out_v),
              (neg_hbm, v_tab, out_n))

    for src_idx, tab, dst in tables:
        idx_copies = []
        for j in range(N_CHUNKS):
            c = pltpu.make_async_copy(
                src_idx.at[pl.ds(base + j * CHUNK, CHUNK)],
                idx_raw.at[j], idx_sem)
            c.start()
            idx_copies.append(c)
        for c in idx_copies:
            c.wait()

        def issue(g, j):
            vec = idx_raw[j, pl.ds(g * 16, 16)]
            for k in range(16):
                pltpu.make_async_copy(
                    tab.at[vec[k]], rows.at[j, g * 16 + k], gat_sem).start()
            return j

        for j in range(N_CHUNKS):
            lax.fori_loop(0, CHUNK // 16, issue, j)

        for j in range(N_CHUNKS):
            pltpu.make_async_copy(
                tab.at[pl.ds(0, CHUNK)], rows.at[j], gat_sem).wait()

        outs = []
        for j in range(N_CHUNKS):
            c = pltpu.make_async_copy(
                rows.at[j], dst.at[pl.ds(base + j * CHUNK, CHUNK)],
                out_sem)
            c.start()
            outs.append(c)
        for c in outs:
            c.wait()


@jax.jit
def kernel(users, items, neg_items, U, V):
    mesh = plsc.VectorSubcoreMesh(core_axis_name="c", subcore_axis_name="s",
                                  num_cores=NUM_CORES,
                                  num_subcores=NUM_SUBCORES)
    out_sd = jax.ShapeDtypeStruct((B, EMB), jnp.float32)
    f = pl.kernel(
        _gather3_body,
        out_type=(out_sd, out_sd, out_sd),
        mesh=mesh,
        compiler_params=pltpu.CompilerParams(use_tc_tiling_on_sc=True),
        scratch_types=[
            pltpu.VMEM((N_CHUNKS, CHUNK), jnp.int32),
            pltpu.VMEM((N_CHUNKS, CHUNK, EMB), jnp.float32),
            pltpu.SemaphoreType.DMA,
            pltpu.SemaphoreType.DMA,
            pltpu.SemaphoreType.DMA,
        ],
    )
    return f(users, items, neg_items, U, V)
